# Initial kernel scaffold; baseline (speedup 1.0000x reference)
import jax
import jax.numpy as jnp
from jax.experimental import pallas as pl


def kernel(indices, table):
    raise NotImplementedError("write your pallas kernel here")


# speedup vs baseline: 3.7243x; 3.7243x over previous
import functools

import jax
import jax.numpy as jnp
from jax import lax
from jax.experimental import pallas as pl
from jax.experimental.pallas import tpu as pltpu
from jax.experimental.pallas import tpu_sc as plsc

VOCAB_SIZE = 1000


def _build_sc_lookup(B, V, D, Dp, NC, NS, L):
    NW = NC * NS
    b_per_w = B // NW
    C = 2560
    assert b_per_w % C == 0
    n_chunks = b_per_w // C

    mesh = plsc.VectorSubcoreMesh(core_axis_name="c", subcore_axis_name="s")

    @functools.partial(
        pl.kernel,
        mesh=mesh,
        compiler_params=pltpu.CompilerParams(use_tc_tiling_on_sc=False),
        out_type=jax.ShapeDtypeStruct((B, Dp), jnp.float32),
        scratch_types=[
            pltpu.VMEM((C,), jnp.int32),
            pltpu.VMEM((C, Dp), jnp.float32),
            pltpu.SemaphoreType.DMA,
        ],
    )
    def sc_lookup(idx_hbm, table_hbm, out_hbm, idx_v, rows_v, sem):
        wid = lax.axis_index("s") * NC + lax.axis_index("c")
        base = wid * b_per_w

        def chunk_body(ci, _):
            off = base + ci * C
            pltpu.sync_copy(idx_hbm.at[pl.ds(off, C)], idx_v)

            def remap(j, _):
                v = idx_v[pl.ds(j * L, L)]
                ok = (v >= 0) & (v < V)
                idx_v[pl.ds(j * L, L)] = jnp.where(ok, v + 1, jnp.zeros_like(v))
                return 0

            lax.fori_loop(0, C // L, remap, 0, unroll=4)

            pltpu.async_copy(table_hbm.at[idx_v], rows_v, sem).wait()
            pltpu.sync_copy(rows_v, out_hbm.at[pl.ds(off, C)])
            return 0

        lax.fori_loop(0, n_chunks, chunk_body, 0)

    return sc_lookup


def kernel(indices, table):
    B = indices.shape[0] * indices.shape[1]
    V = VOCAB_SIZE
    D = table.shape[1]
    Dp = (D + 7) // 8 * 8
    info = plsc.get_sparse_core_info()
    NC, NS, L = info.num_cores, info.num_subcores, info.num_lanes
    sc_lookup = _build_sc_lookup(B, V, D, Dp, NC, NS, L)
    flat_idx = indices.reshape(B)
    table_p = jnp.pad(table, ((0, 0), (0, Dp - D)))
    out = sc_lookup(flat_idx, table_p)
    return out[:, :D].reshape(indices.shape[0], indices.shape[1], D)
